# Initial kernel scaffold; baseline (speedup 1.0000x reference)
import jax
import jax.numpy as jnp
from jax.experimental import pallas as pl


def kernel(x, params, consts):
    raise NotImplementedError("write your pallas kernel here")


# speedup vs baseline: 3.2537x; 3.2537x over previous
import functools
import math

import jax
import jax.numpy as jnp
from jax import lax
from jax.experimental import pallas as pl
from jax.experimental.pallas import tpu as pltpu
from jax.experimental.pallas import tpu_sc as plsc

KNN = 20
NPTS = 2048
NBATCH = 8
N = NBATCH * NPTS
TWO_PI = 2.0 * math.pi

LAYERS = [
    (3, 3, 64, 20, 1),
    (64, 32, 64, 10, 2),
    (64, 32, 128, 7, 3),
    (128, 64, 256, 5, 4),
]


def _gelu(t):
    return 0.5 * t * (1.0 + lax.erf(t / math.sqrt(2.0)))


_KNN_RB = 256


def _knn_body(xr_ref, xc_ref, idx_ref):
    b = pl.program_id(0)
    xr = xr_ref[0]
    xc = xc_ref[0]
    inner = -2.0 * jnp.dot(xr.astype(jnp.bfloat16), xc.astype(jnp.bfloat16),
                           preferred_element_type=jnp.float32)
    xx_r = (xr[:, 0:1] * xr[:, 0:1] + xr[:, 1:2] * xr[:, 1:2]) + xr[:, 2:3] * xr[:, 2:3]
    xx_c = (xc[0:1] * xc[0:1] + xc[1:2] * xc[1:2]) + xc[2:3] * xc[2:3]
    neg = -xx_r - inner - xx_c
    iota = lax.broadcasted_iota(jnp.int32, (_KNN_RB, NPTS), 1)
    cols = []
    for _ in range(KNN):
        m = jnp.max(neg, axis=1, keepdims=True)
        idx = jnp.min(jnp.where(neg >= m, iota, NPTS), axis=1, keepdims=True)
        cols.append(idx)
        neg = jnp.where(iota == idx, -jnp.inf, neg)
    idx_ref[0] = jnp.concatenate(cols, axis=1) + b * NPTS


def _knn_call(xr8, xcT8):
    return pl.pallas_call(
        _knn_body,
        grid=(NBATCH, NPTS // _KNN_RB),
        in_specs=[
            pl.BlockSpec((1, _KNN_RB, 8), lambda b, r: (b, r, 0)),
            pl.BlockSpec((1, 8, NPTS), lambda b, r: (b, 0, 0)),
        ],
        out_specs=pl.BlockSpec((1, _KNN_RB, KNN), lambda b, r: (b, r, 0)),
        out_shape=jax.ShapeDtypeStruct((NBATCH, NPTS, KNN), jnp.int32),
    )(xr8, xcT8)


@functools.lru_cache(maxsize=None)
def _make_sc_gather(nidx, d):
    info = plsc.get_sparse_core_info()
    nw = info.num_cores * info.num_subcores
    per_w = nidx // nw
    ch = 128
    steps = per_w // ch
    assert per_w % ch == 0 and nidx % nw == 0

    mesh = plsc.VectorSubcoreMesh(core_axis_name="c", subcore_axis_name="s")

    @functools.partial(
        pl.kernel,
        mesh=mesh,
        compiler_params=pltpu.CompilerParams(use_tc_tiling_on_sc=False),
        out_type=jax.ShapeDtypeStruct((nidx, d), jnp.float32),
        scratch_types=[
            pltpu.VMEM((ch,), jnp.int32),
            pltpu.VMEM((ch, d), jnp.float32),
            pltpu.SemaphoreType.DMA,
        ],
    )
    def gather(table_hbm, idx_hbm, out_hbm, idx_v, rows_v, sem):
        wid = lax.axis_index("s") * info.num_cores + lax.axis_index("c")
        base = wid * per_w

        def body(t, carry):
            off = base + t * ch
            pltpu.sync_copy(idx_hbm.at[pl.ds(off, ch)], idx_v)
            pltpu.async_copy(table_hbm.at[idx_v], rows_v, sem).wait()
            pltpu.sync_copy(rows_v, out_hbm.at[pl.ds(off, ch)])
            return carry

        lax.fori_loop(0, steps, body, 0)

    return gather


def _sc_gather(table, idx_flat):
    return _make_sc_gather(idx_flat.shape[0], table.shape[1])(table, idx_flat)


_P = 512


def _bf(t):
    return t.astype(jnp.bfloat16).astype(jnp.float32)


def _bdot(a, b):
    return jnp.dot(a.astype(jnp.bfloat16), b.astype(jnp.bfloat16),
                   preferred_element_type=jnp.float32)


def _fourier_feats(f7_i, b8, mlpT, mlpb):
    ff = _bdot(f7_i * TWO_PI, b8)
    sc = jnp.concatenate([jnp.sin(ff), jnp.cos(ff)], axis=1)
    return _bdot(sc, mlpT) + mlpb[0:1, :]


def _pm_rows(f7_list, k8, nn):
    pms = []
    col0 = (lax.broadcasted_iota(jnp.int32, (1, nn), 1) == 0).astype(jnp.float32)
    k8b = k8.astype(jnp.bfloat16)
    for i in range(nn):
        pm = jnp.dot(f7_list[i].astype(jnp.bfloat16), k8b,
                     preferred_element_type=jnp.float32)
        if i == 0:
            pm = pm + col0
        pm = jnp.maximum(pm, 0.0)
        pms.append(pm)
    denom = pms[0]
    for p in pms[1:]:
        denom = denom + p
    inv = 1.0 / (denom + 1e-6)
    return [p * inv for p in pms]


def _layer1_body(gx_ref, b8_ref, mlpT_ref, mlpb_ref, k8_ref, w_ref, cb_ref,
                 f7_ref, out_ref, st_ref):
    nn = 20
    xrep = gx_ref[0][:, 0:3]
    f7s, fcs = [], []
    zero1 = jnp.zeros((_P, 1), jnp.float32)
    for i in range(nn):
        xi = gx_ref[i][:, 0:3]
        xrel = xi - xrep
        sq = (xrel[:, 0:1] * xrel[:, 0:1] + xrel[:, 1:2] * xrel[:, 1:2]) \
            + xrel[:, 2:3] * xrel[:, 2:3]
        nrm = jnp.sqrt(jnp.where(sq > 0, sq, 1.0))
        dis = jnp.where(sq > 0, nrm, 0.0)
        f7 = jnp.concatenate([xrep, xrel, dis, zero1], axis=1)
        f7_ref[i] = f7
        f7s.append(f7)
        fcs.append(xi)
    pms = _pm_rows(f7s, k8_ref[...], nn)
    b8, mlpT, mlpb = b8_ref[...], mlpT_ref[...], mlpb_ref[...]
    for i in range(nn):
        fcs[i] = jnp.concatenate(
            [fcs[i], _fourier_feats(f7s[i], b8, mlpT, mlpb)], axis=1)
    fcb = [_bf(f) for f in fcs]
    pmb = [_bf(p) for p in pms]
    fo_cols = []
    for c in range(6):
        acc = fcb[0][:, c:c + 1] * pmb[0]
        for i in range(1, nn):
            acc = acc + fcb[i][:, c:c + 1] * pmb[i]
        fo_cols.append(acc)
    fo = jnp.concatenate(fo_cols, axis=1)
    out = _bdot(fo, w_ref[...])
    out = out + cb_ref[0:1, :]
    out_ref[...] = out

    @pl.when(pl.program_id(0) == 0)
    def _():
        st_ref[...] = jnp.zeros_like(st_ref)

    st_ref[0:1, :] += jnp.sum(out, axis=0, keepdims=True)
    st_ref[1:2, :] += jnp.sum(out * out, axis=0, keepdims=True)


def _layerN_body(cfg, f7_ref, g_ref, aff_ref, b8_ref, mlpT_ref, mlpb_ref,
                 k8_ref, w_ref, cb_ref, out_ref, st_ref):
    ic, icx, oc, nn, dil = cfg
    c_tot = ic + icx
    sc_row = aff_ref[0:1, :]
    sh_row = aff_ref[1:2, :]
    f7s = [f7_ref[i * dil] for i in range(nn)]
    pms = _pm_rows(f7s, k8_ref[...], nn)
    b8, mlpT, mlpb = b8_ref[...], mlpT_ref[...], mlpb_ref[...]
    fcs = []
    for i in range(nn):
        feats = _gelu(g_ref[i] * sc_row + sh_row)
        fcs.append(jnp.concatenate(
            [feats, _fourier_feats(f7s[i], b8, mlpT, mlpb)], axis=1))
    fcb = [_bf(f) for f in fcs]
    pmb = [_bf(p) for p in pms]
    fo_blocks = []
    for j in range(nn):
        acc = fcb[0] * pmb[0][:, j:j + 1]
        for i in range(1, nn):
            acc = acc + fcb[i] * pmb[i][:, j:j + 1]
        fo_blocks.append(acc)
    fo = jnp.concatenate(fo_blocks, axis=1)
    out = _bdot(fo, w_ref[...])
    out = out + cb_ref[0:1, :]
    out_ref[...] = out

    @pl.when(pl.program_id(0) == 0)
    def _():
        st_ref[...] = jnp.zeros_like(st_ref)

    st_ref[0:1, :] += jnp.sum(out, axis=0, keepdims=True)
    st_ref[1:2, :] += jnp.sum(out * out, axis=0, keepdims=True)


def _layer1_call(gx, b8, mlpT, mlpb, k8, w, cb):
    nn, oc = 20, 64
    grid = (N // _P,)
    small = lambda shape: pl.BlockSpec(shape, lambda i: tuple(0 for _ in shape))
    return pl.pallas_call(
        _layer1_body,
        grid=grid,
        in_specs=[
            pl.BlockSpec((nn, _P, 16), lambda i: (0, i, 0)),
            small(b8.shape), small(mlpT.shape), small(mlpb.shape),
            small(k8.shape), small(w.shape), small(cb.shape),
        ],
        out_specs=[
            pl.BlockSpec((nn, _P, 8), lambda i: (0, i, 0)),
            pl.BlockSpec((_P, oc), lambda i: (i, 0)),
            pl.BlockSpec((8, oc), lambda i: (0, 0)),
        ],
        out_shape=[
            jax.ShapeDtypeStruct((nn, N, 8), jnp.float32),
            jax.ShapeDtypeStruct((N, oc), jnp.float32),
            jax.ShapeDtypeStruct((8, oc), jnp.float32),
        ],
    )(gx, b8, mlpT, mlpb, k8, w, cb)


def _layerN_call(cfg, f7, g, aff, b8, mlpT, mlpb, k8, w, cb):
    ic, icx, oc, nn, dil = cfg
    grid = (N // _P,)
    small = lambda shape: pl.BlockSpec(shape, lambda i: tuple(0 for _ in shape))
    return pl.pallas_call(
        functools.partial(_layerN_body, cfg),
        grid=grid,
        in_specs=[
            pl.BlockSpec((20, _P, 8), lambda i: (0, i, 0)),
            pl.BlockSpec((nn, _P, ic), lambda i: (0, i, 0)),
            small(aff.shape), small(b8.shape), small(mlpT.shape),
            small(mlpb.shape), small(k8.shape), small(w.shape), small(cb.shape),
        ],
        out_specs=[
            pl.BlockSpec((_P, oc), lambda i: (i, 0)),
            pl.BlockSpec((8, oc), lambda i: (0, 0)),
        ],
        out_shape=[
            jax.ShapeDtypeStruct((N, oc), jnp.float32),
            jax.ShapeDtypeStruct((8, oc), jnp.float32),
        ],
    )(f7, g, aff, b8, mlpT, mlpb, k8, w, cb)


def _head1_body(o1, o2, o3, o4, a1, a2, a3, a4, w5_ref, out_ref, st_ref):
    parts = []
    for o, a in ((o1, a1), (o2, a2), (o3, a3), (o4, a4)):
        parts.append(_gelu(o[...] * a[0:1, :] + a[1:2, :]))
    h = jnp.concatenate(parts, axis=1)
    out = _bdot(h, w5_ref[...])
    out_ref[...] = out

    @pl.when(pl.program_id(0) == 0)
    def _():
        st_ref[...] = jnp.zeros_like(st_ref)

    st_ref[0:1, :] += jnp.sum(out, axis=0, keepdims=True)
    st_ref[1:2, :] += jnp.sum(out * out, axis=0, keepdims=True)


def _head1_call(outs, affs, w5T):
    grid = (N // _P,)
    small = lambda shape: pl.BlockSpec(shape, lambda i: tuple(0 for _ in shape))
    emb = w5T.shape[1]
    return pl.pallas_call(
        _head1_body,
        grid=grid,
        in_specs=[pl.BlockSpec((_P, o.shape[1]), lambda i: (i, 0)) for o in outs]
        + [small(a.shape) for a in affs] + [small(w5T.shape)],
        out_specs=[
            pl.BlockSpec((_P, emb), lambda i: (i, 0)),
            pl.BlockSpec((8, emb), lambda i: (0, 0)),
        ],
        out_shape=[
            jax.ShapeDtypeStruct((N, emb), jnp.float32),
            jax.ShapeDtypeStruct((8, emb), jnp.float32),
        ],
    )(*outs, *affs, w5T)


def _head2_body(o5_ref, a5_ref, mx_ref, sm_ref):
    b = pl.program_id(0)
    h = _gelu(o5_ref[...] * a5_ref[0:1, :] + a5_ref[1:2, :])
    mx_ref[pl.ds(b, 1), :] = jnp.max(h, axis=0, keepdims=True)
    sm_ref[pl.ds(b, 1), :] = jnp.mean(h, axis=0, keepdims=True)


def _head2_call(o5, a5):
    emb = o5.shape[1]
    small = lambda shape: pl.BlockSpec(shape, lambda i: tuple(0 for _ in shape))
    return pl.pallas_call(
        _head2_body,
        grid=(NBATCH,),
        in_specs=[pl.BlockSpec((NPTS, emb), lambda b: (b, 0)), small(a5.shape)],
        out_specs=[small((NBATCH, emb)), small((NBATCH, emb))],
        out_shape=[
            jax.ShapeDtypeStruct((NBATCH, emb), jnp.float32),
            jax.ShapeDtypeStruct((NBATCH, emb), jnp.float32),
        ],
    )(o5, a5)


def _bn_rows(t, g, b):
    m = jnp.mean(t, axis=0, keepdims=True)
    v = jnp.mean((t - m) ** 2, axis=0, keepdims=True)
    return (t - m) / jnp.sqrt(v + 1e-5) * g[0:1, :] + b[0:1, :]


def _head3_body(mx_ref, sm_ref, w1_ref, g6_ref, b6_ref, w2_ref, b2_ref,
                g7_ref, b7_ref, w3_ref, b3_ref, out_ref):
    h = jnp.concatenate([mx_ref[...], sm_ref[...]], axis=1)
    t = _bdot(h, w1_ref[...])
    t = _gelu(_bn_rows(t, g6_ref, b6_ref))
    t = _bdot(t, w2_ref[...]) + b2_ref[0:1, :]
    t = _gelu(_bn_rows(t, g7_ref, b7_ref))
    t = _bdot(t, w3_ref[...]) + b3_ref[0:1, :]
    out_ref[...] = t


def _head3_call(mx, sm, w1T, g6, b6, w2T, b2, g7, b7, w3T, b3):
    args = (mx, sm, w1T, g6, b6, w2T, b2, g7, b7, w3T, b3)
    small = lambda shape: pl.BlockSpec(shape, lambda: tuple(0 for _ in shape))
    return pl.pallas_call(
        _head3_body,
        in_specs=[small(a.shape) for a in args],
        out_specs=small((NBATCH, 40)),
        out_shape=jax.ShapeDtypeStruct((NBATCH, 40), jnp.float32),
    )(*args)


def _row8(v):
    return jnp.broadcast_to(v[None, :], (8, v.shape[0]))


def _affine_from_stats(st, g, b):
    mean = st[0] / N
    var = st[1] / N - mean * mean
    scale = g / jnp.sqrt(var + 1e-5)
    shift = b - mean * scale
    return jnp.stack([scale, shift]).astype(jnp.float32)


def _pad_rows8(a):
    r = (-a.shape[0]) % 8
    return jnp.pad(a, ((0, r), (0, 0))) if r else a


def _layer_weights(p, c, cfg):
    ic, icx, oc, nn, dil = cfg
    b8 = jnp.pad(c['B'], ((0, 1), (0, 0)))
    mlpT = p['mlp_w'].T
    mlpb = _row8(p['mlp_b'])
    k8 = jnp.zeros((8, nn), jnp.float32).at[3:6].set(c['kernels'])
    cb = _row8(p['conv_b'])
    return b8, mlpT, mlpb, k8, cb


def kernel(x, params, consts):
    xt = jnp.transpose(x, (0, 2, 1)).reshape(N, 3)
    xt16 = jnp.pad(xt, ((0, 0), (0, 13)))
    xr8 = jnp.pad(jnp.transpose(x, (0, 2, 1)), ((0, 0), (0, 0), (0, 5)))
    xcT8 = jnp.pad(x, ((0, 0), (0, 5), (0, 0)))

    idxg = _knn_call(xr8, xcT8)

    def idx_flat(dil):
        sub = idxg[:, :, ::dil]
        return jnp.transpose(sub, (2, 0, 1)).reshape(-1)

    cfg1 = LAYERS[0]
    b8, mlpT, mlpb, k8, cb = _layer_weights(params['conv1'], consts['conv1'], cfg1)
    w1 = params['conv1']['conv_w'].T
    gx = _sc_gather(xt16, idx_flat(1)).reshape(20, N, 16)
    f7, out1, st1 = _layer1_call(gx, b8, mlpT, mlpb, k8, w1, cb)
    aff1 = _affine_from_stats(st1, params['conv1']['bn_g'], params['conv1']['bn_b'])

    outs = [out1]
    affs = [_pad_rows8(aff1)]
    prev = out1
    for li in range(1, 4):
        cfg = LAYERS[li]
        ic, icx, oc, nn, dil = cfg
        p = params['conv%d' % (li + 1)]
        c = consts['conv%d' % (li + 1)]
        b8, mlpT, mlpb, k8, cb = _layer_weights(p, c, cfg)
        wp = p['conv_w'].reshape(oc, ic + icx, nn).transpose(2, 1, 0)
        wp = wp.reshape(nn * (ic + icx), oc)
        g = _sc_gather(prev, idx_flat(dil)).reshape(nn, N, ic)
        out, st = _layerN_call(cfg, f7, g, affs[-1], b8, mlpT, mlpb, k8, wp, cb)
        aff = _affine_from_stats(st, p['bn_g'], p['bn_b'])
        outs.append(out)
        affs.append(_pad_rows8(aff))
        prev = out

    w5T = params['conv5_w'].T
    out5, st5 = _head1_call(outs, affs, w5T)
    aff5 = _pad_rows8(_affine_from_stats(st5, params['bn5_g'], params['bn5_b']))
    mx, sm = _head2_call(out5, aff5)
    out = _head3_call(
        mx, sm,
        params['lin1_w'].T, _row8(params['bn6_g']), _row8(params['bn6_b']),
        params['lin2_w'].T, _row8(params['lin2_b']),
        _row8(params['bn7_g']), _row8(params['bn7_b']),
        params['lin3_w'].T, _row8(params['lin3_b']),
    )
    return out


# speedup vs baseline: 3.2669x; 1.0041x over previous
import functools
import math

import jax
import jax.numpy as jnp
from jax import lax
from jax.experimental import pallas as pl
from jax.experimental.pallas import tpu as pltpu
from jax.experimental.pallas import tpu_sc as plsc

KNN = 20
NPTS = 2048
NBATCH = 8
N = NBATCH * NPTS
TWO_PI = 2.0 * math.pi

LAYERS = [
    (3, 3, 64, 20, 1),
    (64, 32, 64, 10, 2),
    (64, 32, 128, 7, 3),
    (128, 64, 256, 5, 4),
]


def _gelu(t):
    return 0.5 * t * (1.0 + lax.erf(t / math.sqrt(2.0)))


_KNN_RB = 256


def _knn_body(xr_ref, xc_ref, idx_ref):
    b = pl.program_id(0)
    xr = xr_ref[0]
    xc = xc_ref[0]
    inner = -2.0 * jnp.dot(xr.astype(jnp.bfloat16), xc.astype(jnp.bfloat16),
                           preferred_element_type=jnp.float32)
    xx_r = (xr[:, 0:1] * xr[:, 0:1] + xr[:, 1:2] * xr[:, 1:2]) + xr[:, 2:3] * xr[:, 2:3]
    xx_c = (xc[0:1] * xc[0:1] + xc[1:2] * xc[1:2]) + xc[2:3] * xc[2:3]
    neg = -xx_r - inner - xx_c
    iota = lax.broadcasted_iota(jnp.int32, (_KNN_RB, NPTS), 1)
    cols = []
    for _ in range(KNN):
        m = jnp.max(neg, axis=1, keepdims=True)
        idx = jnp.min(jnp.where(neg >= m, iota, NPTS), axis=1, keepdims=True)
        cols.append(idx)
        neg = jnp.where(iota == idx, -jnp.inf, neg)
    idx_ref[0] = jnp.concatenate(cols, axis=1) + b * NPTS


def _knn_call(xr8, xcT8):
    return pl.pallas_call(
        _knn_body,
        grid=(NBATCH, NPTS // _KNN_RB),
        in_specs=[
            pl.BlockSpec((1, _KNN_RB, 8), lambda b, r: (b, r, 0)),
            pl.BlockSpec((1, 8, NPTS), lambda b, r: (b, 0, 0)),
        ],
        out_specs=pl.BlockSpec((1, _KNN_RB, KNN), lambda b, r: (b, r, 0)),
        out_shape=jax.ShapeDtypeStruct((NBATCH, NPTS, KNN), jnp.int32),
    )(xr8, xcT8)


@functools.lru_cache(maxsize=None)
def _make_sc_gather(nidx, d):
    info = plsc.get_sparse_core_info()
    nw = info.num_cores * info.num_subcores
    per_w = nidx // nw
    ch = 128
    steps = per_w // ch
    assert per_w % ch == 0 and nidx % nw == 0

    mesh = plsc.VectorSubcoreMesh(core_axis_name="c", subcore_axis_name="s")

    @functools.partial(
        pl.kernel,
        mesh=mesh,
        compiler_params=pltpu.CompilerParams(use_tc_tiling_on_sc=False),
        out_type=jax.ShapeDtypeStruct((nidx, d), jnp.float32),
        scratch_types=[
            pltpu.VMEM((ch,), jnp.int32),
            pltpu.VMEM((ch, d), jnp.float32),
            pltpu.SemaphoreType.DMA,
        ],
    )
    def gather(table_hbm, idx_hbm, out_hbm, idx_v, rows_v, sem):
        wid = lax.axis_index("s") * info.num_cores + lax.axis_index("c")
        base = wid * per_w

        def body(t, carry):
            off = base + t * ch
            pltpu.sync_copy(idx_hbm.at[pl.ds(off, ch)], idx_v)
            pltpu.async_copy(table_hbm.at[idx_v], rows_v, sem).wait()
            pltpu.sync_copy(rows_v, out_hbm.at[pl.ds(off, ch)])
            return carry

        lax.fori_loop(0, steps, body, 0)

    return gather


def _sc_gather(table, idx_flat):
    return _make_sc_gather(idx_flat.shape[0], table.shape[1])(table, idx_flat)


_P = 512


def _bf(t):
    return t.astype(jnp.bfloat16).astype(jnp.float32)


def _bdot(a, b):
    return jnp.dot(a.astype(jnp.bfloat16), b.astype(jnp.bfloat16),
                   preferred_element_type=jnp.float32)


def _fourier_feats(f7_i, b8, mlpT, mlpb):
    ff = _bdot(f7_i * TWO_PI, b8)
    sc = jnp.concatenate([jnp.sin(ff), jnp.cos(ff)], axis=1)
    return _bdot(sc, mlpT) + mlpb[0:1, :]


def _pm_rows(f7_list, k8, nn):
    pms = []
    col0 = (lax.broadcasted_iota(jnp.int32, (1, nn), 1) == 0).astype(jnp.float32)
    k8b = k8.astype(jnp.bfloat16)
    for i in range(nn):
        pm = jnp.dot(f7_list[i].astype(jnp.bfloat16), k8b,
                     preferred_element_type=jnp.float32)
        if i == 0:
            pm = pm + col0
        pm = jnp.maximum(pm, 0.0)
        pms.append(pm)
    denom = pms[0]
    for p in pms[1:]:
        denom = denom + p
    inv = 1.0 / (denom + 1e-6)
    return [p * inv for p in pms]


def _layer1_body(gx_ref, b8_ref, mlpT_ref, mlpb_ref, k8_ref, w_ref, cb_ref,
                 f7_ref, out_ref, st_ref):
    nn = 20
    xrep = gx_ref[0][:, 0:3]
    f7s, fcs = [], []
    zero1 = jnp.zeros((_P, 1), jnp.float32)
    for i in range(nn):
        xi = gx_ref[i][:, 0:3]
        xrel = xi - xrep
        sq = (xrel[:, 0:1] * xrel[:, 0:1] + xrel[:, 1:2] * xrel[:, 1:2]) \
            + xrel[:, 2:3] * xrel[:, 2:3]
        nrm = jnp.sqrt(jnp.where(sq > 0, sq, 1.0))
        dis = jnp.where(sq > 0, nrm, 0.0)
        f7 = jnp.concatenate([xrep, xrel, dis, zero1], axis=1)
        f7s.append(f7)
        fcs.append(xi)
    f7_ref[...] = jnp.concatenate(f7s, axis=1)
    pms = _pm_rows(f7s, k8_ref[...], nn)
    b8, mlpT, mlpb = b8_ref[...], mlpT_ref[...], mlpb_ref[...]
    for i in range(nn):
        fcs[i] = jnp.concatenate(
            [fcs[i], _fourier_feats(f7s[i], b8, mlpT, mlpb)], axis=1)
    fcb = [_bf(f) for f in fcs]
    pmb = [_bf(p) for p in pms]
    fo_cols = []
    for c in range(6):
        acc = fcb[0][:, c:c + 1] * pmb[0]
        for i in range(1, nn):
            acc = acc + fcb[i][:, c:c + 1] * pmb[i]
        fo_cols.append(acc)
    fo = jnp.concatenate(fo_cols, axis=1)
    out = _bdot(fo, w_ref[...])
    out = out + cb_ref[0:1, :]
    out_ref[...] = out

    @pl.when(pl.program_id(0) == 0)
    def _():
        st_ref[...] = jnp.zeros_like(st_ref)

    st_ref[0:1, :] += jnp.sum(out, axis=0, keepdims=True)
    st_ref[1:2, :] += jnp.sum(out * out, axis=0, keepdims=True)


def _layerN_body(cfg, f7_ref, g_ref, aff_ref, b8_ref, mlpT_ref, mlpb_ref,
                 k8_ref, w_ref, cb_ref, out_ref, st_ref):
    ic, icx, oc, nn, dil = cfg
    c_tot = ic + icx
    sc_row = aff_ref[0:1, :]
    sh_row = aff_ref[1:2, :]
    f7s = [f7_ref[:, i * dil * 8:(i * dil + 1) * 8] for i in range(nn)]
    pms = _pm_rows(f7s, k8_ref[...], nn)
    b8, mlpT, mlpb = b8_ref[...], mlpT_ref[...], mlpb_ref[...]
    fcs = []
    for i in range(nn):
        feats = _gelu(g_ref[i] * sc_row + sh_row)
        fcs.append(jnp.concatenate(
            [feats, _fourier_feats(f7s[i], b8, mlpT, mlpb)], axis=1))
    fcb = [_bf(f) for f in fcs]
    pmb = [_bf(p) for p in pms]
    fo_blocks = []
    for j in range(nn):
        acc = fcb[0] * pmb[0][:, j:j + 1]
        for i in range(1, nn):
            acc = acc + fcb[i] * pmb[i][:, j:j + 1]
        fo_blocks.append(acc)
    fo = jnp.concatenate(fo_blocks, axis=1)
    out = _bdot(fo, w_ref[...])
    out = out + cb_ref[0:1, :]
    out_ref[...] = out

    @pl.when(pl.program_id(0) == 0)
    def _():
        st_ref[...] = jnp.zeros_like(st_ref)

    st_ref[0:1, :] += jnp.sum(out, axis=0, keepdims=True)
    st_ref[1:2, :] += jnp.sum(out * out, axis=0, keepdims=True)


def _layer1_call(gx, b8, mlpT, mlpb, k8, w, cb):
    nn, oc = 20, 64
    grid = (N // _P,)
    small = lambda shape: pl.BlockSpec(shape, lambda i: tuple(0 for _ in shape))
    return pl.pallas_call(
        _layer1_body,
        grid=grid,
        in_specs=[
            pl.BlockSpec((nn, _P, 16), lambda i: (0, i, 0)),
            small(b8.shape), small(mlpT.shape), small(mlpb.shape),
            small(k8.shape), small(w.shape), small(cb.shape),
        ],
        out_specs=[
            pl.BlockSpec((_P, nn * 8), lambda i: (i, 0)),
            pl.BlockSpec((_P, oc), lambda i: (i, 0)),
            pl.BlockSpec((8, oc), lambda i: (0, 0)),
        ],
        out_shape=[
            jax.ShapeDtypeStruct((N, nn * 8), jnp.float32),
            jax.ShapeDtypeStruct((N, oc), jnp.float32),
            jax.ShapeDtypeStruct((8, oc), jnp.float32),
        ],
    )(gx, b8, mlpT, mlpb, k8, w, cb)


def _layerN_call(cfg, f7, g, aff, b8, mlpT, mlpb, k8, w, cb):
    ic, icx, oc, nn, dil = cfg
    grid = (N // _P,)
    small = lambda shape: pl.BlockSpec(shape, lambda i: tuple(0 for _ in shape))
    return pl.pallas_call(
        functools.partial(_layerN_body, cfg),
        grid=grid,
        in_specs=[
            pl.BlockSpec((_P, 160), lambda i: (i, 0)),
            pl.BlockSpec((nn, _P, ic), lambda i: (0, i, 0)),
            small(aff.shape), small(b8.shape), small(mlpT.shape),
            small(mlpb.shape), small(k8.shape), small(w.shape), small(cb.shape),
        ],
        out_specs=[
            pl.BlockSpec((_P, oc), lambda i: (i, 0)),
            pl.BlockSpec((8, oc), lambda i: (0, 0)),
        ],
        out_shape=[
            jax.ShapeDtypeStruct((N, oc), jnp.float32),
            jax.ShapeDtypeStruct((8, oc), jnp.float32),
        ],
    )(f7, g, aff, b8, mlpT, mlpb, k8, w, cb)


def _head1_body(o1, o2, o3, o4, a1, a2, a3, a4, w5_ref, out_ref, st_ref):
    parts = []
    for o, a in ((o1, a1), (o2, a2), (o3, a3), (o4, a4)):
        parts.append(_gelu(o[...] * a[0:1, :] + a[1:2, :]))
    h = jnp.concatenate(parts, axis=1)
    out = _bdot(h, w5_ref[...])
    out_ref[...] = out

    @pl.when(pl.program_id(0) == 0)
    def _():
        st_ref[...] = jnp.zeros_like(st_ref)

    st_ref[0:1, :] += jnp.sum(out, axis=0, keepdims=True)
    st_ref[1:2, :] += jnp.sum(out * out, axis=0, keepdims=True)


def _head1_call(outs, affs, w5T):
    grid = (N // _P,)
    small = lambda shape: pl.BlockSpec(shape, lambda i: tuple(0 for _ in shape))
    emb = w5T.shape[1]
    return pl.pallas_call(
        _head1_body,
        grid=grid,
        in_specs=[pl.BlockSpec((_P, o.shape[1]), lambda i: (i, 0)) for o in outs]
        + [small(a.shape) for a in affs] + [small(w5T.shape)],
        out_specs=[
            pl.BlockSpec((_P, emb), lambda i: (i, 0)),
            pl.BlockSpec((8, emb), lambda i: (0, 0)),
        ],
        out_shape=[
            jax.ShapeDtypeStruct((N, emb), jnp.float32),
            jax.ShapeDtypeStruct((8, emb), jnp.float32),
        ],
    )(*outs, *affs, w5T)


def _head2_body(o5_ref, a5_ref, mx_ref, sm_ref):
    b = pl.program_id(0)
    h = _gelu(o5_ref[...] * a5_ref[0:1, :] + a5_ref[1:2, :])
    mx_ref[pl.ds(b, 1), :] = jnp.max(h, axis=0, keepdims=True)
    sm_ref[pl.ds(b, 1), :] = jnp.mean(h, axis=0, keepdims=True)


def _head2_call(o5, a5):
    emb = o5.shape[1]
    small = lambda shape: pl.BlockSpec(shape, lambda i: tuple(0 for _ in shape))
    return pl.pallas_call(
        _head2_body,
        grid=(NBATCH,),
        in_specs=[pl.BlockSpec((NPTS, emb), lambda b: (b, 0)), small(a5.shape)],
        out_specs=[small((NBATCH, emb)), small((NBATCH, emb))],
        out_shape=[
            jax.ShapeDtypeStruct((NBATCH, emb), jnp.float32),
            jax.ShapeDtypeStruct((NBATCH, emb), jnp.float32),
        ],
    )(o5, a5)


def _bn_rows(t, g, b):
    m = jnp.mean(t, axis=0, keepdims=True)
    v = jnp.mean((t - m) ** 2, axis=0, keepdims=True)
    return (t - m) / jnp.sqrt(v + 1e-5) * g[0:1, :] + b[0:1, :]


def _head3_body(mx_ref, sm_ref, w1_ref, g6_ref, b6_ref, w2_ref, b2_ref,
                g7_ref, b7_ref, w3_ref, b3_ref, out_ref):
    h = jnp.concatenate([mx_ref[...], sm_ref[...]], axis=1)
    t = _bdot(h, w1_ref[...])
    t = _gelu(_bn_rows(t, g6_ref, b6_ref))
    t = _bdot(t, w2_ref[...]) + b2_ref[0:1, :]
    t = _gelu(_bn_rows(t, g7_ref, b7_ref))
    t = _bdot(t, w3_ref[...]) + b3_ref[0:1, :]
    out_ref[...] = t


def _head3_call(mx, sm, w1T, g6, b6, w2T, b2, g7, b7, w3T, b3):
    args = (mx, sm, w1T, g6, b6, w2T, b2, g7, b7, w3T, b3)
    small = lambda shape: pl.BlockSpec(shape, lambda: tuple(0 for _ in shape))
    return pl.pallas_call(
        _head3_body,
        in_specs=[small(a.shape) for a in args],
        out_specs=small((NBATCH, 40)),
        out_shape=jax.ShapeDtypeStruct((NBATCH, 40), jnp.float32),
    )(*args)


def _row8(v):
    return jnp.broadcast_to(v[None, :], (8, v.shape[0]))


def _affine_from_stats(st, g, b):
    mean = st[0] / N
    var = st[1] / N - mean * mean
    scale = g / jnp.sqrt(var + 1e-5)
    shift = b - mean * scale
    return jnp.stack([scale, shift]).astype(jnp.float32)


def _pad_rows8(a):
    r = (-a.shape[0]) % 8
    return jnp.pad(a, ((0, r), (0, 0))) if r else a


def _layer_weights(p, c, cfg):
    ic, icx, oc, nn, dil = cfg
    b8 = jnp.pad(c['B'], ((0, 1), (0, 0)))
    mlpT = p['mlp_w'].T
    mlpb = _row8(p['mlp_b'])
    k8 = jnp.zeros((8, nn), jnp.float32).at[3:6].set(c['kernels'])
    cb = _row8(p['conv_b'])
    return b8, mlpT, mlpb, k8, cb


def kernel(x, params, consts):
    xt = jnp.transpose(x, (0, 2, 1)).reshape(N, 3)
    xt16 = jnp.pad(xt, ((0, 0), (0, 13)))
    xr8 = jnp.pad(jnp.transpose(x, (0, 2, 1)), ((0, 0), (0, 0), (0, 5)))
    xcT8 = jnp.pad(x, ((0, 0), (0, 5), (0, 0)))

    idxg = _knn_call(xr8, xcT8)

    def idx_flat(dil):
        sub = idxg[:, :, ::dil]
        return jnp.transpose(sub, (2, 0, 1)).reshape(-1)

    cfg1 = LAYERS[0]
    b8, mlpT, mlpb, k8, cb = _layer_weights(params['conv1'], consts['conv1'], cfg1)
    w1 = params['conv1']['conv_w'].T
    gx = _sc_gather(xt16, idx_flat(1)).reshape(20, N, 16)
    f7, out1, st1 = _layer1_call(gx, b8, mlpT, mlpb, k8, w1, cb)
    aff1 = _affine_from_stats(st1, params['conv1']['bn_g'], params['conv1']['bn_b'])

    outs = [out1]
    affs = [_pad_rows8(aff1)]
    prev = out1
    for li in range(1, 4):
        cfg = LAYERS[li]
        ic, icx, oc, nn, dil = cfg
        p = params['conv%d' % (li + 1)]
        c = consts['conv%d' % (li + 1)]
        b8, mlpT, mlpb, k8, cb = _layer_weights(p, c, cfg)
        wp = p['conv_w'].reshape(oc, ic + icx, nn).transpose(2, 1, 0)
        wp = wp.reshape(nn * (ic + icx), oc)
        g = _sc_gather(prev, idx_flat(dil)).reshape(nn, N, ic)
        out, st = _layerN_call(cfg, f7, g, affs[-1], b8, mlpT, mlpb, k8, wp, cb)
        aff = _affine_from_stats(st, p['bn_g'], p['bn_b'])
        outs.append(out)
        affs.append(_pad_rows8(aff))
        prev = out

    w5T = params['conv5_w'].T
    out5, st5 = _head1_call(outs, affs, w5T)
    aff5 = _pad_rows8(_affine_from_stats(st5, params['bn5_g'], params['bn5_b']))
    mx, sm = _head2_call(out5, aff5)
    out = _head3_call(
        mx, sm,
        params['lin1_w'].T, _row8(params['bn6_g']), _row8(params['bn6_b']),
        params['lin2_w'].T, _row8(params['lin2_b']),
        _row8(params['bn7_g']), _row8(params['bn7_b']),
        params['lin3_w'].T, _row8(params['lin3_b']),
    )
    return out
